# Initial kernel scaffold; baseline (speedup 1.0000x reference)
import functools

import jax
import jax.numpy as jnp
from jax import lax
from jax.experimental import pallas as pl
from jax.experimental.pallas import tpu as pltpu
from jax.experimental.pallas import tpu_sc as plsc

N_NODES = 10000
N_EDGES = 320000
D = 128

NC = 2
NS = 16
NW = NC * NS
NPAD = 10240
EPAD = 327680
EPW = EPAD // NW
CHUNK = 128
NCHUNK = EPW // CHUNK
RPT = NPAD // NS


def _sc_aggregate(y, src, dst, zrows, zdeg, with_deg):
    mesh = plsc.VectorSubcoreMesh(core_axis_name="c", subcore_axis_name="s")
    out_type = [jax.ShapeDtypeStruct((NC, NPAD, D), jnp.float32)]
    if with_deg:
        out_type.append(jax.ShapeDtypeStruct((NC, NPAD, 16), jnp.float32))

    scratch = [
        pltpu.VMEM_SHARED((NPAD, D), jnp.float32),
        pltpu.VMEM((1, CHUNK), jnp.int32),
        pltpu.VMEM((1, CHUNK), jnp.int32),
        pltpu.VMEM((CHUNK, D), jnp.float32),
        pltpu.SemaphoreType.DMA,
    ]
    if with_deg:
        scratch += [
            pltpu.VMEM_SHARED((NPAD, 16), jnp.float32),
            pltpu.VMEM((CHUNK, 16), jnp.float32),
        ]

    @functools.partial(
        pl.kernel, out_type=tuple(out_type), mesh=mesh,
        scratch_types=scratch)
    def k(y_hbm, src_hbm, dst_hbm, zrows_hbm, zdeg_hbm, *refs):
        if with_deg:
            (acc_out, deg_out, acc_sp, srcb, dstb, rowsb, sem,
             deg_sp, onesb) = refs
        else:
            acc_out, acc_sp, srcb, dstb, rowsb, sem = refs
        cid = lax.axis_index("c")
        sid = lax.axis_index("s")
        wid = cid * NS + sid

        rslice = pl.ds(sid * RPT, RPT)
        pltpu.sync_copy(zrows_hbm.at[rslice], acc_sp.at[rslice])
        if with_deg:
            pltpu.sync_copy(zdeg_hbm.at[rslice], deg_sp.at[rslice])

            @pl.loop(0, CHUNK)
            def _(i):
                onesb[i] = jnp.ones((16,), jnp.float32)

        plsc.subcore_barrier()

        base = wid * EPW

        @pl.loop(0, NCHUNK)
        def _(c):
            off = base + c * CHUNK
            pltpu.sync_copy(src_hbm.at[pl.ds(off, CHUNK)], srcb.at[0])
            pltpu.sync_copy(dst_hbm.at[pl.ds(off, CHUNK)], dstb.at[0])
            pltpu.async_copy(y_hbm.at[srcb.at[0]], rowsb, sem).wait()
            pltpu.sync_copy(rowsb, acc_sp.at[dstb.at[0]], add=True)
            if with_deg:
                pltpu.sync_copy(onesb, deg_sp.at[dstb.at[0]], add=True)

        plsc.subcore_barrier()

        pltpu.sync_copy(acc_sp.at[rslice], acc_out.at[cid, rslice])
        if with_deg:
            pltpu.sync_copy(deg_sp.at[rslice], deg_out.at[cid, rslice])

    return k(y, src, dst, zrows, zdeg)


ROWBLK = 512
GRID = NPAD // ROWBLK


def _combine_body(relu, acc_ref, deg_ref, x_ref, wl_ref, b_ref, wr_ref,
                  o_ref):
    s = acc_ref[0] + acc_ref[1]
    d = deg_ref[0, :, 0:1] + deg_ref[1, :, 0:1]
    agg = s / jnp.maximum(d, 1.0)
    h = (jnp.dot(agg, wl_ref[...].T, preferred_element_type=jnp.float32)
         + b_ref[...]
         + jnp.dot(x_ref[...], wr_ref[...].T,
                   preferred_element_type=jnp.float32))
    o_ref[...] = jnp.maximum(h, 0.0) if relu else h


def _combine(acc, deg, x, W_l, b_l, W_r, relu):
    return pl.pallas_call(
        functools.partial(_combine_body, relu),
        grid=(GRID,),
        in_specs=[
            pl.BlockSpec((NC, ROWBLK, D), lambda i: (0, i, 0)),
            pl.BlockSpec((NC, ROWBLK, 16), lambda i: (0, i, 0)),
            pl.BlockSpec((ROWBLK, D), lambda i: (i, 0)),
            pl.BlockSpec((D, D), lambda i: (0, 0)),
            pl.BlockSpec((1, D), lambda i: (0, 0)),
            pl.BlockSpec((D, D), lambda i: (0, 0)),
        ],
        out_specs=pl.BlockSpec((ROWBLK, D), lambda i: (i, 0)),
        out_shape=jax.ShapeDtypeStruct((NPAD, D), jnp.float32),
    )(acc, deg, x, W_l, b_l, W_r)


def kernel(x, edge_index, W1_l, b1_l, W1_r, W2_l, b2_l, W2_r):
    src = edge_index[0].astype(jnp.int32)
    dst = edge_index[1].astype(jnp.int32)
    pad_e = EPAD - N_EDGES
    src_p = jnp.concatenate([src, jnp.zeros((pad_e,), jnp.int32)])
    dst_p = jnp.concatenate([dst, jnp.full((pad_e,), N_NODES, jnp.int32)])
    xp = jnp.zeros((NPAD, D), jnp.float32).at[:N_NODES].set(x)
    zrows = jnp.zeros((NPAD, D), jnp.float32)
    zdeg = jnp.zeros((NPAD, 16), jnp.float32)
    b1 = b1_l.reshape(1, D)
    b2 = b2_l.reshape(1, D)

    acc1, deg = _sc_aggregate(xp, src_p, dst_p, zrows, zdeg, with_deg=True)
    h = _combine(acc1, deg, xp, W1_l, b1, W1_r, relu=True)
    (acc2,) = _sc_aggregate(h, src_p, dst_p, zrows, zdeg, with_deg=False)
    out = _combine(acc2, deg, h, W2_l, b2, W2_r, relu=False)
    return out[:N_NODES]


# speedup vs baseline: 3.0592x; 3.0592x over previous
import functools

import jax
import jax.numpy as jnp
from jax import lax
from jax.experimental import pallas as pl
from jax.experimental.pallas import tpu as pltpu
from jax.experimental.pallas import tpu_sc as plsc

N_NODES = 10000
N_EDGES = 320000
D = 128
DH = D // 2

NC = 2
NS = 16
NPAD = 10240
EPAD = 327680
EPW = EPAD // NS
CHUNK = 128
NCHUNK = EPW // CHUNK
RPT = NPAD // NS
ZB = 128
NZB = RPT // ZB


def _sc_aggregate(yt, src2, dst, with_deg):
    mesh = plsc.VectorSubcoreMesh(core_axis_name="c", subcore_axis_name="s")
    out_type = [jax.ShapeDtypeStruct((NC * NPAD, DH), jnp.float32)]
    if with_deg:
        out_type.append(jax.ShapeDtypeStruct((NPAD, 16), jnp.float32))

    scratch = [
        pltpu.VMEM_SHARED((NPAD, DH), jnp.float32),
        pltpu.VMEM((CHUNK,), jnp.int32),
        pltpu.VMEM((CHUNK,), jnp.int32),
        pltpu.VMEM((CHUNK, DH), jnp.float32),
        pltpu.VMEM((ZB, DH), jnp.float32),
        pltpu.SemaphoreType.DMA,
    ]
    if with_deg:
        scratch += [
            pltpu.VMEM_SHARED((NPAD, 16), jnp.float32),
            pltpu.VMEM((CHUNK, 16), jnp.float32),
            pltpu.VMEM((ZB, 16), jnp.float32),
        ]

    @functools.partial(
        pl.kernel, out_type=tuple(out_type), mesh=mesh,
        scratch_types=scratch,
        compiler_params=pltpu.CompilerParams(use_tc_tiling_on_sc=False))
    def k(yt_hbm, src2_hbm, dst_hbm, *refs):
        if with_deg:
            (acc_out, deg_out, acc_sp, srcb, dstb, rowsb, stage, sem,
             deg_sp, onesb, dstage) = refs
        else:
            acc_out, acc_sp, srcb, dstb, rowsb, stage, sem = refs
        cid = lax.axis_index("c")
        sid = lax.axis_index("s")
        do_deg = with_deg

        @pl.loop(0, ZB)
        def _(r):
            @pl.loop(0, DH, step=16)
            def _(f):
                stage[r, pl.ds(f, 16)] = jnp.zeros((16,), jnp.float32)

        @pl.loop(0, NZB)
        def _(z):
            pltpu.sync_copy(stage, acc_sp.at[pl.ds(sid * RPT + z * ZB, ZB)])

        if do_deg:
            @pl.loop(0, ZB)
            def _(r):
                dstage[r] = jnp.zeros((16,), jnp.float32)

            @pl.loop(0, NZB)
            def _(z):
                pltpu.sync_copy(dstage,
                                deg_sp.at[pl.ds(sid * RPT + z * ZB, ZB)])

            @pl.loop(0, CHUNK)
            def _(i):
                onesb[i] = jnp.ones((16,), jnp.float32)

        plsc.subcore_barrier()

        ibase = cid * EPAD + sid * EPW
        dbase = sid * EPW

        @pl.loop(0, NCHUNK)
        def _(c):
            pltpu.sync_copy(src2_hbm.at[pl.ds(ibase + c * CHUNK, CHUNK)],
                            srcb)
            pltpu.sync_copy(dst_hbm.at[pl.ds(dbase + c * CHUNK, CHUNK)],
                            dstb)
            pltpu.async_copy(yt_hbm.at[srcb], rowsb, sem).wait()
            pltpu.sync_copy(rowsb, acc_sp.at[dstb], add=True)
            if do_deg:
                @pl.when(cid == 0)
                def _():
                    pltpu.sync_copy(onesb, deg_sp.at[dstb], add=True)

        plsc.subcore_barrier()

        @pl.loop(0, NZB)
        def _(z):
            sp = pl.ds(sid * RPT + z * ZB, ZB)
            ob = pl.ds(cid * NPAD + sid * RPT + z * ZB, ZB)
            pltpu.sync_copy(acc_sp.at[sp], stage)
            pltpu.sync_copy(stage, acc_out.at[ob])
            if do_deg:
                @pl.when(cid == 0)
                def _():
                    pltpu.sync_copy(deg_sp.at[sp], dstage)
                    pltpu.sync_copy(dstage, deg_out.at[sp])

    return k(yt, src2, dst)


ROWBLK = 512
GRID = NPAD // ROWBLK


def _combine_body(relu, acca_ref, accb_ref, deg_ref, x_ref, wl_ref, b_ref,
                  wr_ref, o_ref):
    s = jnp.concatenate([acca_ref[...], accb_ref[...]], axis=1)
    d = jnp.maximum(deg_ref[:, 0:1], 1.0)
    agg = s / d
    h = (jnp.dot(agg, wl_ref[...].T, preferred_element_type=jnp.float32)
         + b_ref[...]
         + jnp.dot(x_ref[...], wr_ref[...].T,
                   preferred_element_type=jnp.float32))
    o_ref[...] = jnp.maximum(h, 0.0) if relu else h


def _combine(acc, deg, x, W_l, b_l, W_r, relu):
    acca = acc[:NPAD]
    accb = acc[NPAD:]
    return pl.pallas_call(
        functools.partial(_combine_body, relu),
        grid=(GRID,),
        in_specs=[
            pl.BlockSpec((ROWBLK, DH), lambda i: (i, 0)),
            pl.BlockSpec((ROWBLK, DH), lambda i: (i, 0)),
            pl.BlockSpec((ROWBLK, 16), lambda i: (i, 0)),
            pl.BlockSpec((ROWBLK, D), lambda i: (i, 0)),
            pl.BlockSpec((D, D), lambda i: (0, 0)),
            pl.BlockSpec((1, D), lambda i: (0, 0)),
            pl.BlockSpec((D, D), lambda i: (0, 0)),
        ],
        out_specs=pl.BlockSpec((ROWBLK, D), lambda i: (i, 0)),
        out_shape=jax.ShapeDtypeStruct((NPAD, D), jnp.float32),
    )(acca, accb, deg, x, W_l, b_l, W_r)


def kernel(x, edge_index, W1_l, b1_l, W1_r, W2_l, b2_l, W2_r):
    src = edge_index[0].astype(jnp.int32)
    dst = edge_index[1].astype(jnp.int32)
    pad_e = EPAD - N_EDGES
    src_p = jnp.concatenate([src, jnp.zeros((pad_e,), jnp.int32)])
    dst_p = jnp.concatenate([dst, jnp.full((pad_e,), N_NODES, jnp.int32)])
    src2 = jnp.concatenate([src_p, src_p + NPAD])
    xp = jnp.zeros((NPAD, D), jnp.float32).at[:N_NODES].set(x)
    b1 = b1_l.reshape(1, D)
    b2 = b2_l.reshape(1, D)

    def feature_major(y):
        return jnp.concatenate([y[:, :DH], y[:, DH:]], axis=0)

    acc1, deg = _sc_aggregate(feature_major(xp), src2, dst_p, with_deg=True)
    h = _combine(acc1, deg, xp, W1_l, b1, W1_r, relu=True)
    (acc2,) = _sc_aggregate(feature_major(h), src2, dst_p, with_deg=False)
    out = _combine(acc2, deg, h, W2_l, b2, W2_r, relu=False)
    return out[:N_NODES]


# speedup vs baseline: 3.6427x; 1.1907x over previous
import functools

import jax
import jax.numpy as jnp
from jax import lax
from jax.experimental import pallas as pl
from jax.experimental.pallas import tpu as pltpu
from jax.experimental.pallas import tpu_sc as plsc

N_NODES = 10000
N_EDGES = 320000
D = 128
DH = D // 2

NC = 2
NS = 16
NPAD = 10240
EPAD = 327680
EPW = EPAD // NS
CHUNK = 128
NPAIR = EPW // (2 * CHUNK)
RPT = NPAD // NS
ZB = 128
NZB = RPT // ZB


def _sc_aggregate(yt, src2, dst, with_deg):
    mesh = plsc.VectorSubcoreMesh(core_axis_name="c", subcore_axis_name="s")
    out_type = [jax.ShapeDtypeStruct((NC * NPAD, DH), jnp.float32)]
    if with_deg:
        out_type.append(jax.ShapeDtypeStruct((NPAD, 16), jnp.float32))

    scratch = [
        pltpu.VMEM_SHARED((NPAD, DH), jnp.float32),
        pltpu.VMEM((CHUNK,), jnp.int32),
        pltpu.VMEM((CHUNK,), jnp.int32),
        pltpu.VMEM((CHUNK,), jnp.int32),
        pltpu.VMEM((CHUNK,), jnp.int32),
        pltpu.VMEM((CHUNK, DH), jnp.float32),
        pltpu.VMEM((CHUNK, DH), jnp.float32),
        pltpu.VMEM((ZB, DH), jnp.float32),
        pltpu.SemaphoreType.DMA,
        pltpu.SemaphoreType.DMA,
    ]
    if with_deg:
        scratch += [
            pltpu.VMEM_SHARED((NPAD, 16), jnp.float32),
            pltpu.VMEM((CHUNK, 16), jnp.float32),
            pltpu.VMEM((ZB, 16), jnp.float32),
        ]

    @functools.partial(
        pl.kernel, out_type=tuple(out_type), mesh=mesh,
        scratch_types=scratch,
        compiler_params=pltpu.CompilerParams(use_tc_tiling_on_sc=False))
    def k(yt_hbm, src2_hbm, dst_hbm, *refs):
        if with_deg:
            (acc_out, deg_out, acc_sp, srcA, dstA, srcB, dstB, rowsA,
             rowsB, stage, gsem, ssem, deg_sp, onesb, dstage) = refs
        else:
            (acc_out, acc_sp, srcA, dstA, srcB, dstB, rowsA, rowsB,
             stage, gsem, ssem) = refs
        cid = lax.axis_index("c")
        sid = lax.axis_index("s")
        do_deg = with_deg

        @pl.loop(0, ZB)
        def _(r):
            @pl.loop(0, DH, step=16)
            def _(f):
                stage[r, pl.ds(f, 16)] = jnp.zeros((16,), jnp.float32)

        @pl.loop(0, NZB)
        def _(z):
            pltpu.sync_copy(stage, acc_sp.at[pl.ds(sid * RPT + z * ZB, ZB)])

        if do_deg:
            @pl.loop(0, ZB)
            def _(r):
                dstage[r] = jnp.zeros((16,), jnp.float32)

            @pl.loop(0, NZB)
            def _(z):
                pltpu.sync_copy(dstage,
                                deg_sp.at[pl.ds(sid * RPT + z * ZB, ZB)])

            @pl.loop(0, CHUNK)
            def _(i):
                onesb[i] = jnp.ones((16,), jnp.float32)

        plsc.subcore_barrier()

        ibase = cid * EPAD + sid * EPW
        dbase = sid * EPW

        def deg_add(dref):
            if do_deg:
                @pl.when(cid == 0)
                def _():
                    pltpu.sync_copy(onesb, deg_sp.at[dref], add=True)

        @pl.loop(0, NPAIR)
        def _(t):
            offA = 2 * t * CHUNK
            offB = offA + CHUNK
            pltpu.sync_copy(src2_hbm.at[pl.ds(ibase + offA, CHUNK)], srcA)
            pltpu.sync_copy(dst_hbm.at[pl.ds(dbase + offA, CHUNK)], dstA)
            gh = pltpu.async_copy(yt_hbm.at[srcA], rowsA, gsem)
            pltpu.sync_copy(src2_hbm.at[pl.ds(ibase + offB, CHUNK)], srcB)
            pltpu.sync_copy(dst_hbm.at[pl.ds(dbase + offB, CHUNK)], dstB)
            gh.wait()
            sh = pltpu.async_copy(rowsA, acc_sp.at[dstA], ssem, add=True)
            gh2 = pltpu.async_copy(yt_hbm.at[srcB], rowsB, gsem)
            deg_add(dstA)
            gh2.wait()
            sh.wait()
            pltpu.sync_copy(rowsB, acc_sp.at[dstB], add=True)
            deg_add(dstB)

        plsc.subcore_barrier()

        @pl.loop(0, NZB)
        def _(z):
            sp = pl.ds(sid * RPT + z * ZB, ZB)
            ob = pl.ds(cid * NPAD + sid * RPT + z * ZB, ZB)
            pltpu.sync_copy(acc_sp.at[sp], stage)
            pltpu.sync_copy(stage, acc_out.at[ob])
            if do_deg:
                @pl.when(cid == 0)
                def _():
                    pltpu.sync_copy(deg_sp.at[sp], dstage)
                    pltpu.sync_copy(dstage, deg_out.at[sp])

    return k(yt, src2, dst)


ROWBLK = 512
GRID = NPAD // ROWBLK


def _combine_body(relu, acca_ref, accb_ref, deg_ref, x_ref, wl_ref, b_ref,
                  wr_ref, o_ref):
    s = jnp.concatenate([acca_ref[...], accb_ref[...]], axis=1)
    d = jnp.maximum(deg_ref[:, 0:1], 1.0)
    agg = s / d
    h = (jnp.dot(agg, wl_ref[...].T, preferred_element_type=jnp.float32)
         + b_ref[...]
         + jnp.dot(x_ref[...], wr_ref[...].T,
                   preferred_element_type=jnp.float32))
    o_ref[...] = jnp.maximum(h, 0.0) if relu else h


def _combine(acc, deg, x, W_l, b_l, W_r, relu):
    acca = acc[:NPAD]
    accb = acc[NPAD:]
    return pl.pallas_call(
        functools.partial(_combine_body, relu),
        grid=(GRID,),
        in_specs=[
            pl.BlockSpec((ROWBLK, DH), lambda i: (i, 0)),
            pl.BlockSpec((ROWBLK, DH), lambda i: (i, 0)),
            pl.BlockSpec((ROWBLK, 16), lambda i: (i, 0)),
            pl.BlockSpec((ROWBLK, D), lambda i: (i, 0)),
            pl.BlockSpec((D, D), lambda i: (0, 0)),
            pl.BlockSpec((1, D), lambda i: (0, 0)),
            pl.BlockSpec((D, D), lambda i: (0, 0)),
        ],
        out_specs=pl.BlockSpec((ROWBLK, D), lambda i: (i, 0)),
        out_shape=jax.ShapeDtypeStruct((NPAD, D), jnp.float32),
    )(acca, accb, deg, x, W_l, b_l, W_r)


def kernel(x, edge_index, W1_l, b1_l, W1_r, W2_l, b2_l, W2_r):
    src = edge_index[0].astype(jnp.int32)
    dst = edge_index[1].astype(jnp.int32)
    pad_e = EPAD - N_EDGES
    src_p = jnp.concatenate([src, jnp.zeros((pad_e,), jnp.int32)])
    dst_p = jnp.concatenate([dst, jnp.full((pad_e,), N_NODES, jnp.int32)])
    src2 = jnp.concatenate([src_p, src_p + NPAD])
    xp = jnp.zeros((NPAD, D), jnp.float32).at[:N_NODES].set(x)
    b1 = b1_l.reshape(1, D)
    b2 = b2_l.reshape(1, D)

    def feature_major(y):
        return jnp.concatenate([y[:, :DH], y[:, DH:]], axis=0)

    acc1, deg = _sc_aggregate(feature_major(xp), src2, dst_p, with_deg=True)
    h = _combine(acc1, deg, xp, W1_l, b1, W1_r, relu=True)
    (acc2,) = _sc_aggregate(feature_major(h), src2, dst_p, with_deg=False)
    out = _combine(acc2, deg, h, W2_l, b2, W2_r, relu=False)
    return out[:N_NODES]


# speedup vs baseline: 3.7376x; 1.0261x over previous
import functools

import jax
import jax.numpy as jnp
from jax import lax
from jax.experimental import pallas as pl
from jax.experimental.pallas import tpu as pltpu
from jax.experimental.pallas import tpu_sc as plsc

N_NODES = 10000
N_EDGES = 320000
D = 128
DH = D // 2

NC = 2
NS = 16
NPAD = 10240
EPAD = 327680
EPW = EPAD // NS
CHUNK = 128
NPAIR = EPW // (2 * CHUNK)
RPT = NPAD // NS
ZB = 128
NZB = RPT // ZB


def _sc_aggregate(yt, src2, dst, with_deg):
    mesh = plsc.VectorSubcoreMesh(core_axis_name="c", subcore_axis_name="s")
    out_type = [jax.ShapeDtypeStruct((NC * NPAD, DH), jnp.float32)]
    if with_deg:
        out_type.append(jax.ShapeDtypeStruct((NPAD, 16), jnp.float32))

    scratch = [
        pltpu.VMEM_SHARED((NPAD, DH), jnp.float32),
        pltpu.VMEM((CHUNK,), jnp.int32),
        pltpu.VMEM((CHUNK,), jnp.int32),
        pltpu.VMEM((CHUNK,), jnp.int32),
        pltpu.VMEM((CHUNK,), jnp.int32),
        pltpu.VMEM((CHUNK, DH), jnp.float32),
        pltpu.VMEM((CHUNK, DH), jnp.float32),
        pltpu.VMEM((ZB, DH), jnp.float32),
        pltpu.SemaphoreType.DMA,
        pltpu.SemaphoreType.DMA,
        pltpu.SemaphoreType.DMA,
    ]
    if with_deg:
        scratch += [
            pltpu.VMEM_SHARED((NPAD, 16), jnp.float32),
            pltpu.VMEM((CHUNK, 16), jnp.float32),
            pltpu.VMEM((ZB, 16), jnp.float32),
        ]

    @functools.partial(
        pl.kernel, out_type=tuple(out_type), mesh=mesh,
        scratch_types=scratch,
        compiler_params=pltpu.CompilerParams(use_tc_tiling_on_sc=False))
    def k(yt_hbm, src2_hbm, dst_hbm, *refs):
        if with_deg:
            (acc_out, deg_out, acc_sp, srcA, dstA, srcB, dstB, rowsA,
             rowsB, stage, gsem, gsem2, ssem, deg_sp, onesb, dstage) = refs
        else:
            (acc_out, acc_sp, srcA, dstA, srcB, dstB, rowsA, rowsB,
             stage, gsem, gsem2, ssem) = refs
        cid = lax.axis_index("c")
        sid = lax.axis_index("s")
        do_deg = with_deg

        @pl.loop(0, ZB)
        def _(r):
            @pl.loop(0, DH, step=16)
            def _(f):
                stage[r, pl.ds(f, 16)] = jnp.zeros((16,), jnp.float32)

        @pl.loop(0, NZB)
        def _(z):
            pltpu.sync_copy(stage, acc_sp.at[pl.ds(sid * RPT + z * ZB, ZB)])

        if do_deg:
            @pl.loop(0, ZB)
            def _(r):
                dstage[r] = jnp.zeros((16,), jnp.float32)

            @pl.loop(0, NZB)
            def _(z):
                pltpu.sync_copy(dstage,
                                deg_sp.at[pl.ds(sid * RPT + z * ZB, ZB)])

            @pl.loop(0, CHUNK)
            def _(i):
                onesb[i] = jnp.ones((16,), jnp.float32)

        plsc.subcore_barrier()

        ibase = cid * EPAD + sid * EPW
        dbase = sid * EPW

        def deg_add(dref):
            if do_deg:
                @pl.when(cid == 0)
                def _():
                    pltpu.sync_copy(onesb, deg_sp.at[dref], add=True)

        @pl.loop(0, NPAIR)
        def _(t):
            offA = 2 * t * CHUNK
            offB = offA + CHUNK
            pltpu.sync_copy(src2_hbm.at[pl.ds(ibase + offA, CHUNK)], srcA)
            pltpu.sync_copy(dst_hbm.at[pl.ds(dbase + offA, CHUNK)], dstA)
            gh = pltpu.async_copy(yt_hbm.at[srcA], rowsA, gsem)
            pltpu.sync_copy(src2_hbm.at[pl.ds(ibase + offB, CHUNK)], srcB)
            pltpu.sync_copy(dst_hbm.at[pl.ds(dbase + offB, CHUNK)], dstB)
            gh2 = pltpu.async_copy(yt_hbm.at[srcB], rowsB, gsem2)
            gh.wait()
            sh = pltpu.async_copy(rowsA, acc_sp.at[dstA], ssem, add=True)
            deg_add(dstA)
            gh2.wait()
            sh.wait()
            pltpu.sync_copy(rowsB, acc_sp.at[dstB], add=True)
            deg_add(dstB)

        plsc.subcore_barrier()

        @pl.loop(0, NZB)
        def _(z):
            sp = pl.ds(sid * RPT + z * ZB, ZB)
            ob = pl.ds(cid * NPAD + sid * RPT + z * ZB, ZB)
            pltpu.sync_copy(acc_sp.at[sp], stage)
            pltpu.sync_copy(stage, acc_out.at[ob])
            if do_deg:
                @pl.when(cid == 0)
                def _():
                    pltpu.sync_copy(deg_sp.at[sp], dstage)
                    pltpu.sync_copy(dstage, deg_out.at[sp])

    return k(yt, src2, dst)


ROWBLK = 512
GRID = NPAD // ROWBLK


def _combine_body(relu, acca_ref, accb_ref, deg_ref, x_ref, wl_ref, b_ref,
                  wr_ref, o_ref):
    s = jnp.concatenate([acca_ref[...], accb_ref[...]], axis=1)
    d = jnp.maximum(deg_ref[:, 0:1], 1.0)
    agg = s / d
    h = (jnp.dot(agg, wl_ref[...].T, preferred_element_type=jnp.float32)
         + b_ref[...]
         + jnp.dot(x_ref[...], wr_ref[...].T,
                   preferred_element_type=jnp.float32))
    o_ref[...] = jnp.maximum(h, 0.0) if relu else h


def _combine(acc, deg, x, W_l, b_l, W_r, relu):
    acca = acc[:NPAD]
    accb = acc[NPAD:]
    return pl.pallas_call(
        functools.partial(_combine_body, relu),
        grid=(GRID,),
        in_specs=[
            pl.BlockSpec((ROWBLK, DH), lambda i: (i, 0)),
            pl.BlockSpec((ROWBLK, DH), lambda i: (i, 0)),
            pl.BlockSpec((ROWBLK, 16), lambda i: (i, 0)),
            pl.BlockSpec((ROWBLK, D), lambda i: (i, 0)),
            pl.BlockSpec((D, D), lambda i: (0, 0)),
            pl.BlockSpec((1, D), lambda i: (0, 0)),
            pl.BlockSpec((D, D), lambda i: (0, 0)),
        ],
        out_specs=pl.BlockSpec((ROWBLK, D), lambda i: (i, 0)),
        out_shape=jax.ShapeDtypeStruct((NPAD, D), jnp.float32),
    )(acca, accb, deg, x, W_l, b_l, W_r)


def kernel(x, edge_index, W1_l, b1_l, W1_r, W2_l, b2_l, W2_r):
    src = edge_index[0].astype(jnp.int32)
    dst = edge_index[1].astype(jnp.int32)
    pad_e = EPAD - N_EDGES
    src_p = jnp.concatenate([src, jnp.zeros((pad_e,), jnp.int32)])
    dst_p = jnp.concatenate([dst, jnp.full((pad_e,), N_NODES, jnp.int32)])
    src2 = jnp.concatenate([src_p, src_p + NPAD])
    xp = jnp.zeros((NPAD, D), jnp.float32).at[:N_NODES].set(x)
    b1 = b1_l.reshape(1, D)
    b2 = b2_l.reshape(1, D)

    def feature_major(y):
        return jnp.concatenate([y[:, :DH], y[:, DH:]], axis=0)

    acc1, deg = _sc_aggregate(feature_major(xp), src2, dst_p, with_deg=True)
    h = _combine(acc1, deg, xp, W1_l, b1, W1_r, relu=True)
    (acc2,) = _sc_aggregate(feature_major(h), src2, dst_p, with_deg=False)
    out = _combine(acc2, deg, h, W2_l, b2, W2_r, relu=False)
    return out[:N_NODES]


# speedup vs baseline: 4.0932x; 1.0951x over previous
import functools

import jax
import jax.numpy as jnp
from jax import lax
from jax.experimental import pallas as pl
from jax.experimental.pallas import tpu as pltpu
from jax.experimental.pallas import tpu_sc as plsc

N_NODES = 10000
N_EDGES = 320000
D = 128
DH = D // 2

NC = 2
NS = 16
NPAD = 10240
EPAD = 327680
EPW = EPAD // NS
CHUNK = 128
IDXB = 1024
GPB = IDXB // CHUNK
NBLKS = EPW // IDXB
RPT = NPAD // NS
ZB = 128
NZB = RPT // ZB


def _sc_aggregate(yt, src2, dst, with_deg):
    mesh = plsc.VectorSubcoreMesh(core_axis_name="c", subcore_axis_name="s")
    out_type = [jax.ShapeDtypeStruct((NC * NPAD, DH), jnp.float32)]
    if with_deg:
        out_type.append(jax.ShapeDtypeStruct((NPAD, 16), jnp.float32))

    scratch = [
        pltpu.VMEM_SHARED((NPAD, DH), jnp.float32),
        pltpu.VMEM((IDXB,), jnp.int32),
        pltpu.VMEM((CHUNK,), jnp.int32),
        pltpu.VMEM((CHUNK,), jnp.int32),
        pltpu.VMEM((CHUNK, DH), jnp.float32),
        pltpu.VMEM((CHUNK, DH), jnp.float32),
        pltpu.VMEM((ZB, DH), jnp.float32),
        pltpu.SemaphoreType.DMA,
        pltpu.SemaphoreType.DMA,
        pltpu.SemaphoreType.DMA,
    ]
    if with_deg:
        scratch += [
            pltpu.VMEM_SHARED((NPAD, 16), jnp.float32),
            pltpu.VMEM((CHUNK, 16), jnp.float32),
            pltpu.VMEM((ZB, 16), jnp.float32),
        ]

    @functools.partial(
        pl.kernel, out_type=tuple(out_type), mesh=mesh,
        scratch_types=scratch,
        compiler_params=pltpu.CompilerParams(use_tc_tiling_on_sc=False))
    def k(yt_hbm, src2_hbm, dst_hbm, *refs):
        if with_deg:
            (acc_out, deg_out, acc_sp, srcBig, dstA, dstB, rowsA, rowsB,
             stage, gsemA, gsemB, ssem, deg_sp, onesb, dstage) = refs
        else:
            (acc_out, acc_sp, srcBig, dstA, dstB, rowsA, rowsB, stage,
             gsemA, gsemB, ssem) = refs
        cid = lax.axis_index("c")
        sid = lax.axis_index("s")
        do_deg = with_deg

        @pl.loop(0, ZB)
        def _(r):
            @pl.loop(0, DH, step=16)
            def _(f):
                stage[r, pl.ds(f, 16)] = jnp.zeros((16,), jnp.float32)

        @pl.loop(0, NZB)
        def _(z):
            pltpu.sync_copy(stage, acc_sp.at[pl.ds(sid * RPT + z * ZB, ZB)])

        if do_deg:
            @pl.loop(0, ZB)
            def _(r):
                dstage[r] = jnp.zeros((16,), jnp.float32)

            @pl.loop(0, NZB)
            def _(z):
                pltpu.sync_copy(dstage,
                                deg_sp.at[pl.ds(sid * RPT + z * ZB, ZB)])

            @pl.loop(0, CHUNK)
            def _(i):
                onesb[i] = jnp.ones((16,), jnp.float32)

        plsc.subcore_barrier()

        ibase = cid * EPAD + sid * EPW
        dbase = sid * EPW

        def deg_add(dref):
            if do_deg:
                @pl.when(cid == 0)
                def _():
                    pltpu.sync_copy(onesb, deg_sp.at[dref], add=True)

        @pl.loop(0, NBLKS)
        def _(b):
            pltpu.sync_copy(src2_hbm.at[pl.ds(ibase + b * IDXB, IDXB)],
                            srcBig)
            dchunk0 = dbase + b * IDXB
            for pp in range(GPB // 2):
                kA = 2 * pp
                kB = kA + 1
                pltpu.sync_copy(
                    dst_hbm.at[pl.ds(dchunk0 + kA * CHUNK, CHUNK)], dstA)
                gh = pltpu.async_copy(
                    yt_hbm.at[srcBig.at[pl.ds(kA * CHUNK, CHUNK)]],
                    rowsA, gsemA)
                pltpu.sync_copy(
                    dst_hbm.at[pl.ds(dchunk0 + kB * CHUNK, CHUNK)], dstB)
                gh2 = pltpu.async_copy(
                    yt_hbm.at[srcBig.at[pl.ds(kB * CHUNK, CHUNK)]],
                    rowsB, gsemB)
                gh.wait()
                sh = pltpu.async_copy(rowsA, acc_sp.at[dstA], ssem,
                                      add=True)
                deg_add(dstA)
                gh2.wait()
                sh.wait()
                pltpu.sync_copy(rowsB, acc_sp.at[dstB], add=True)
                deg_add(dstB)

        plsc.subcore_barrier()

        @pl.loop(0, NZB)
        def _(z):
            sp = pl.ds(sid * RPT + z * ZB, ZB)
            ob = pl.ds(cid * NPAD + sid * RPT + z * ZB, ZB)
            pltpu.sync_copy(acc_sp.at[sp], stage)
            pltpu.sync_copy(stage, acc_out.at[ob])
            if do_deg:
                @pl.when(cid == 0)
                def _():
                    pltpu.sync_copy(deg_sp.at[sp], dstage)
                    pltpu.sync_copy(dstage, deg_out.at[sp])

    return k(yt, src2, dst)


ROWBLK = 512
GRID = NPAD // ROWBLK


def _combine_body(relu, acca_ref, accb_ref, deg_ref, x_ref, wl_ref, b_ref,
                  wr_ref, o_ref):
    s = jnp.concatenate([acca_ref[...], accb_ref[...]], axis=1)
    d = jnp.maximum(deg_ref[:, 0:1], 1.0)
    agg = s / d
    h = (jnp.dot(agg, wl_ref[...].T, preferred_element_type=jnp.float32)
         + b_ref[...]
         + jnp.dot(x_ref[...], wr_ref[...].T,
                   preferred_element_type=jnp.float32))
    o_ref[...] = jnp.maximum(h, 0.0) if relu else h


def _combine(acc, deg, x, W_l, b_l, W_r, relu):
    acca = acc[:NPAD]
    accb = acc[NPAD:]
    return pl.pallas_call(
        functools.partial(_combine_body, relu),
        grid=(GRID,),
        in_specs=[
            pl.BlockSpec((ROWBLK, DH), lambda i: (i, 0)),
            pl.BlockSpec((ROWBLK, DH), lambda i: (i, 0)),
            pl.BlockSpec((ROWBLK, 16), lambda i: (i, 0)),
            pl.BlockSpec((ROWBLK, D), lambda i: (i, 0)),
            pl.BlockSpec((D, D), lambda i: (0, 0)),
            pl.BlockSpec((1, D), lambda i: (0, 0)),
            pl.BlockSpec((D, D), lambda i: (0, 0)),
        ],
        out_specs=pl.BlockSpec((ROWBLK, D), lambda i: (i, 0)),
        out_shape=jax.ShapeDtypeStruct((NPAD, D), jnp.float32),
    )(acca, accb, deg, x, W_l, b_l, W_r)


def kernel(x, edge_index, W1_l, b1_l, W1_r, W2_l, b2_l, W2_r):
    src = edge_index[0].astype(jnp.int32)
    dst = edge_index[1].astype(jnp.int32)
    pad_e = EPAD - N_EDGES
    src_p = jnp.concatenate([src, jnp.zeros((pad_e,), jnp.int32)])
    dst_p = jnp.concatenate([dst, jnp.full((pad_e,), N_NODES, jnp.int32)])
    src2 = jnp.concatenate([src_p, src_p + NPAD])
    xp = jnp.zeros((NPAD, D), jnp.float32).at[:N_NODES].set(x)
    b1 = b1_l.reshape(1, D)
    b2 = b2_l.reshape(1, D)

    def feature_major(y):
        return jnp.concatenate([y[:, :DH], y[:, DH:]], axis=0)

    acc1, deg = _sc_aggregate(feature_major(xp), src2, dst_p, with_deg=True)
    h = _combine(acc1, deg, xp, W1_l, b1, W1_r, relu=True)
    (acc2,) = _sc_aggregate(feature_major(h), src2, dst_p, with_deg=False)
    out = _combine(acc2, deg, h, W2_l, b2, W2_r, relu=False)
    return out[:N_NODES]


# speedup vs baseline: 4.1500x; 1.0139x over previous
import functools

import jax
import jax.numpy as jnp
from jax import lax
from jax.experimental import pallas as pl
from jax.experimental.pallas import tpu as pltpu
from jax.experimental.pallas import tpu_sc as plsc

N_NODES = 10000
N_EDGES = 320000
D = 128
DH = D // 2

NC = 2
NS = 16
NPAD = 10240
EPAD = 327680
EPW = EPAD // NS
CHUNK = 128
IDXB = 1024
GPB = IDXB // CHUNK
NBLKS = EPW // IDXB
RPT = NPAD // NS
ZB = 128
NZB = RPT // ZB


def _sc_aggregate(yt, src2, dst, with_deg):
    mesh = plsc.VectorSubcoreMesh(core_axis_name="c", subcore_axis_name="s")
    out_type = [jax.ShapeDtypeStruct((NC * NPAD, DH), jnp.float32)]
    if with_deg:
        out_type.append(jax.ShapeDtypeStruct((NPAD, 16), jnp.float32))

    scratch = [
        pltpu.VMEM_SHARED((NPAD, DH), jnp.float32),
        pltpu.VMEM((IDXB,), jnp.int32),
        pltpu.VMEM((IDXB,), jnp.int32),
        pltpu.VMEM((CHUNK,), jnp.int32),
        pltpu.VMEM((CHUNK,), jnp.int32),
        pltpu.VMEM((CHUNK, DH), jnp.float32),
        pltpu.VMEM((CHUNK, DH), jnp.float32),
        pltpu.VMEM((ZB, DH), jnp.float32),
        pltpu.SemaphoreType.DMA,
        pltpu.SemaphoreType.DMA,
        pltpu.SemaphoreType.DMA,
    ]
    if with_deg:
        scratch += [
            pltpu.VMEM_SHARED((NPAD, 16), jnp.float32),
            pltpu.VMEM((CHUNK, 16), jnp.float32),
            pltpu.VMEM((ZB, 16), jnp.float32),
        ]

    @functools.partial(
        pl.kernel, out_type=tuple(out_type), mesh=mesh,
        scratch_types=scratch,
        compiler_params=pltpu.CompilerParams(use_tc_tiling_on_sc=False))
    def k(yt_hbm, src2_hbm, dst_hbm, *refs):
        if with_deg:
            (acc_out, deg_out, acc_sp, srcBig, dstBig, dstA, dstB, rowsA,
             rowsB, stage, gsemA, gsemB, ssem,
             deg_sp, onesb, dstage) = refs
        else:
            (acc_out, acc_sp, srcBig, dstBig, dstA, dstB, rowsA, rowsB,
             stage, gsemA, gsemB, ssem) = refs
        cid = lax.axis_index("c")
        sid = lax.axis_index("s")
        do_deg = with_deg

        @pl.loop(0, ZB)
        def _(r):
            @pl.loop(0, DH, step=16)
            def _(f):
                stage[r, pl.ds(f, 16)] = jnp.zeros((16,), jnp.float32)

        @pl.loop(0, NZB)
        def _(z):
            pltpu.sync_copy(stage, acc_sp.at[pl.ds(sid * RPT + z * ZB, ZB)])

        if do_deg:
            @pl.loop(0, ZB)
            def _(r):
                dstage[r] = jnp.zeros((16,), jnp.float32)

            @pl.loop(0, NZB)
            def _(z):
                pltpu.sync_copy(dstage,
                                deg_sp.at[pl.ds(sid * RPT + z * ZB, ZB)])

            @pl.loop(0, CHUNK)
            def _(i):
                onesb[i] = jnp.ones((16,), jnp.float32)

        plsc.subcore_barrier()

        ibase = cid * EPAD + sid * EPW
        dbase = sid * EPW

        def deg_add(dref):
            if do_deg:
                @pl.when(cid == 0)
                def _():
                    pltpu.sync_copy(onesb, deg_sp.at[dref], add=True)

        @pl.loop(0, NBLKS)
        def _(b):
            pltpu.sync_copy(src2_hbm.at[pl.ds(ibase + b * IDXB, IDXB)],
                            srcBig)
            pltpu.sync_copy(dst_hbm.at[pl.ds(dbase + b * IDXB, IDXB)],
                            dstBig)

            def load_dst(kk, dref):
                for j in range(CHUNK // 16):
                    dref[pl.ds(j * 16, 16)] = (
                        dstBig[pl.ds(kk * CHUNK + j * 16, 16)])

            for pp in range(GPB // 2):
                kA = 2 * pp
                kB = kA + 1
                load_dst(kA, dstA)
                gh = pltpu.async_copy(
                    yt_hbm.at[srcBig.at[pl.ds(kA * CHUNK, CHUNK)]],
                    rowsA, gsemA)
                load_dst(kB, dstB)
                gh2 = pltpu.async_copy(
                    yt_hbm.at[srcBig.at[pl.ds(kB * CHUNK, CHUNK)]],
                    rowsB, gsemB)
                gh.wait()
                sh = pltpu.async_copy(rowsA, acc_sp.at[dstA], ssem,
                                      add=True)
                deg_add(dstA)
                gh2.wait()
                sh.wait()
                pltpu.sync_copy(rowsB, acc_sp.at[dstB], add=True)
                deg_add(dstB)

        plsc.subcore_barrier()

        @pl.loop(0, NZB)
        def _(z):
            sp = pl.ds(sid * RPT + z * ZB, ZB)
            ob = pl.ds(cid * NPAD + sid * RPT + z * ZB, ZB)
            pltpu.sync_copy(acc_sp.at[sp], stage)
            pltpu.sync_copy(stage, acc_out.at[ob])
            if do_deg:
                @pl.when(cid == 0)
                def _():
                    pltpu.sync_copy(deg_sp.at[sp], dstage)
                    pltpu.sync_copy(dstage, deg_out.at[sp])

    return k(yt, src2, dst)


ROWBLK = 512
GRID = NPAD // ROWBLK


def _combine_body(relu, acca_ref, accb_ref, deg_ref, x_ref, wl_ref, b_ref,
                  wr_ref, o_ref):
    s = jnp.concatenate([acca_ref[...], accb_ref[...]], axis=1)
    d = jnp.maximum(deg_ref[:, 0:1], 1.0)
    agg = s / d
    h = (jnp.dot(agg, wl_ref[...].T, preferred_element_type=jnp.float32)
         + b_ref[...]
         + jnp.dot(x_ref[...], wr_ref[...].T,
                   preferred_element_type=jnp.float32))
    o_ref[...] = jnp.maximum(h, 0.0) if relu else h


def _combine(acc, deg, x, W_l, b_l, W_r, relu):
    acca = acc[:NPAD]
    accb = acc[NPAD:]
    return pl.pallas_call(
        functools.partial(_combine_body, relu),
        grid=(GRID,),
        in_specs=[
            pl.BlockSpec((ROWBLK, DH), lambda i: (i, 0)),
            pl.BlockSpec((ROWBLK, DH), lambda i: (i, 0)),
            pl.BlockSpec((ROWBLK, 16), lambda i: (i, 0)),
            pl.BlockSpec((ROWBLK, D), lambda i: (i, 0)),
            pl.BlockSpec((D, D), lambda i: (0, 0)),
            pl.BlockSpec((1, D), lambda i: (0, 0)),
            pl.BlockSpec((D, D), lambda i: (0, 0)),
        ],
        out_specs=pl.BlockSpec((ROWBLK, D), lambda i: (i, 0)),
        out_shape=jax.ShapeDtypeStruct((NPAD, D), jnp.float32),
    )(acca, accb, deg, x, W_l, b_l, W_r)


def kernel(x, edge_index, W1_l, b1_l, W1_r, W2_l, b2_l, W2_r):
    src = edge_index[0].astype(jnp.int32)
    dst = edge_index[1].astype(jnp.int32)
    pad_e = EPAD - N_EDGES
    src_p = jnp.concatenate([src, jnp.zeros((pad_e,), jnp.int32)])
    dst_p = jnp.concatenate([dst, jnp.full((pad_e,), N_NODES, jnp.int32)])
    src2 = jnp.concatenate([src_p, src_p + NPAD])
    xp = jnp.zeros((NPAD, D), jnp.float32).at[:N_NODES].set(x)
    b1 = b1_l.reshape(1, D)
    b2 = b2_l.reshape(1, D)

    def feature_major(y):
        return jnp.concatenate([y[:, :DH], y[:, DH:]], axis=0)

    acc1, deg = _sc_aggregate(feature_major(xp), src2, dst_p, with_deg=True)
    h = _combine(acc1, deg, xp, W1_l, b1, W1_r, relu=True)
    (acc2,) = _sc_aggregate(feature_major(h), src2, dst_p, with_deg=False)
    out = _combine(acc2, deg, h, W2_l, b2, W2_r, relu=False)
    return out[:N_NODES]
